# Initial kernel scaffold; baseline (speedup 1.0000x reference)
import jax
import jax.numpy as jnp
from jax.experimental import pallas as pl


def kernel(traj_location, mat2, vector, traj_length, emb_su, emb_sl, emb_tu, emb_tl):
    raise NotImplementedError("write your pallas kernel here")


# speedup vs baseline: 19.5110x; 19.5110x over previous
import functools

import jax
import jax.numpy as jnp
from jax.experimental import pallas as pl


def _body(L, ML, E, ex_su, ex_sl, ex_tu, ex_tl,
          loc_ref, dt_ref, tlb_ref, mat2_ref,
          su_ref, sl_ref, tu_ref, tue_ref, out_ref):
    R = out_ref.shape[0]
    f32 = jnp.float32

    l_idx = jax.lax.broadcasted_iota(jnp.int32, (R, 1), 0) % L
    valid = (l_idx < tlb_ref[...]).astype(f32)
    dt = dt_ref[...]

    col = jax.lax.broadcasted_iota(jnp.int32, (R, ML), 1)
    oh = jnp.where(loc_ref[...] - 1 == col, f32(1.0), f32(0.0)) * valid

    m_of_j = jax.lax.broadcasted_iota(jnp.int32, (ML, ML * E), 1) // E
    m_row = jax.lax.broadcasted_iota(jnp.int32, (ML, ML * E), 0)
    M = (m_of_j == m_row).astype(f32)
    e_of_j = jax.lax.broadcasted_iota(jnp.int32, (E, ML * E), 1) % E
    e_row = jax.lax.broadcasted_iota(jnp.int32, (E, ML * E), 0)
    P = (e_of_j == e_row).astype(f32)

    mat2exp = jnp.dot(mat2_ref[...], M, preferred_element_type=f32)
    dsexp = jnp.dot(oh, mat2exp, preferred_element_type=f32)

    def sel(ref):
        return valid * ref[1:2, :] + (f32(1.0) - valid) * ref[0:1, :]

    s_sl = sel(sl_ref)
    s_su = sel(su_ref)
    s_tl = sel(tu_ref)
    s_tu = sel(tue_ref)

    C = (s_su - s_sl) * f32(1.0 / (ex_su - ex_sl))
    A = s_sl + (s_tl * (ex_tu - dt) + s_tu * (dt - ex_tl)) * f32(
        1.0 / (ex_tu - ex_tl))

    a_exp = jnp.dot(A, P, preferred_element_type=f32)
    c_exp = jnp.dot(C, P, preferred_element_type=f32)
    out_ref[...] = a_exp + dsexp * c_exp


def kernel(traj_location, mat2, vector, traj_length, emb_su, emb_sl, emb_tu, emb_tl):
    B, L = traj_location.shape
    ML = mat2.shape[0]
    E = emb_su.shape[1]
    EX_SU, EX_SL, EX_TU, EX_TL = 100.0, 0.0, 24.0, 0.0

    rows = B * L
    R = 1280
    assert rows % R == 0 and R % L == 0

    loc2 = traj_location.reshape(rows, 1).astype(jnp.int32)
    dt2 = vector.reshape(rows, 1)
    tlb = jnp.broadcast_to(traj_length[:, None].astype(jnp.int32),
                           (B, L)).reshape(rows, 1)

    body = functools.partial(_body, L, ML, E, EX_SU, EX_SL, EX_TU, EX_TL)
    row_spec = pl.BlockSpec((R, 1), lambda i: (i, 0))
    full = lambda shape: pl.BlockSpec(shape, lambda i: (0, 0))

    out = pl.pallas_call(
        body,
        grid=(rows // R,),
        in_specs=[
            row_spec,
            row_spec,
            row_spec,
            full((ML, ML)),
            full((2, E)),
            full((2, E)),
            full((2, E)),
            full((2, E)),
        ],
        out_specs=pl.BlockSpec((R, ML * E), lambda i: (i, 0)),
        out_shape=jax.ShapeDtypeStruct((rows, ML * E), jnp.float32),
    )(loc2, dt2, tlb, mat2, emb_su, emb_sl, emb_tl, emb_tu)

    return out.reshape(B, L, ML, E)


# speedup vs baseline: 226.6302x; 11.6155x over previous
import functools

import jax
import jax.numpy as jnp
from jax.experimental import pallas as pl


def _body(B, ML, E, ex_su, ex_sl, ex_tu, ex_tl,
          locT_ref, dtT_ref, tl_ref, mat2t_ref,
          suT_ref, slT_ref, tlT_ref, tuT_ref, out_ref):
    f32 = jnp.float32
    l = pl.program_id(0)

    loc = locT_ref[0]
    dt = dtT_ref[0]
    valid = jnp.where(l < tl_ref[...], f32(1.0), f32(0.0))

    row = jax.lax.broadcasted_iota(jnp.int32, (ML, B), 0)
    ohT = jnp.where(loc - 1 == row, f32(1.0), f32(0.0)) * valid

    dsT = jnp.dot(mat2t_ref[...], ohT, preferred_element_type=f32)

    j_m = jax.lax.broadcasted_iota(jnp.int32, (ML * E, ML), 0) // E
    m_col = jax.lax.broadcasted_iota(jnp.int32, (ML * E, ML), 1)
    MT = (j_m == m_col).astype(f32)
    j_e = jax.lax.broadcasted_iota(jnp.int32, (ML * E, E), 0) % E
    e_col = jax.lax.broadcasted_iota(jnp.int32, (ML * E, E), 1)
    PT = (j_e == e_col).astype(f32)

    def sel(ref):
        return ref[:, 1:2] * valid + ref[:, 0:1] * (f32(1.0) - valid)

    s_sl = sel(slT_ref)
    s_su = sel(suT_ref)
    s_tl = sel(tlT_ref)
    s_tu = sel(tuT_ref)

    cT = (s_su - s_sl) * f32(1.0 / (ex_su - ex_sl))
    aT = s_sl + (s_tl * (ex_tu - dt) + s_tu * (dt - ex_tl)) * f32(
        1.0 / (ex_tu - ex_tl))

    ds_exp = jnp.dot(MT, dsT, preferred_element_type=f32)
    a_exp = jnp.dot(PT, aT, preferred_element_type=f32)
    c_exp = jnp.dot(PT, cT, preferred_element_type=f32)
    out_ref[...] = a_exp + ds_exp * c_exp


def kernel(traj_location, mat2, vector, traj_length, emb_su, emb_sl, emb_tu, emb_tl):
    B, L = traj_location.shape
    ML = mat2.shape[0]
    E = emb_su.shape[1]
    EX_SU, EX_SL, EX_TU, EX_TL = 100.0, 0.0, 24.0, 0.0

    locT = traj_location.T.reshape(L, 1, B).astype(jnp.int32)
    dtT = vector.T.reshape(L, 1, B)
    tl_row = traj_length.reshape(1, B).astype(jnp.int32)
    mat2t = mat2.T

    body = functools.partial(_body, B, ML, E, EX_SU, EX_SL, EX_TU, EX_TL)
    col_spec = pl.BlockSpec((1, 1, B), lambda l: (l, 0, 0))
    full = lambda shape: pl.BlockSpec(shape, lambda l: (0, 0))

    out = pl.pallas_call(
        body,
        grid=(L,),
        in_specs=[
            col_spec,
            col_spec,
            full((1, B)),
            full((ML, ML)),
            full((E, 2)),
            full((E, 2)),
            full((E, 2)),
            full((E, 2)),
        ],
        out_specs=pl.BlockSpec((ML * E, B), lambda l: (l, 0)),
        out_shape=jax.ShapeDtypeStruct((L * ML * E, B), jnp.float32),
    )(locT, dtT, tl_row, mat2t, emb_su.T, emb_sl.T, emb_tl.T, emb_tu.T)

    return out.reshape(L, ML, E, B).transpose(3, 0, 1, 2)


# speedup vs baseline: 271.7657x; 1.1992x over previous
import functools

import jax
import jax.numpy as jnp
from jax.experimental import pallas as pl
from jax.experimental.pallas import tpu as pltpu


def _tdot(a, b):
    return jax.lax.dot_general(a, b, (((0,), (0,)), ((), ())),
                               preferred_element_type=jnp.float32)


def _body(B, L, ML, E, ex_su, ex_sl, ex_tu, ex_tl,
          loc_ref, dt_ref, tl_ref, mat2_ref,
          su_ref, sl_ref, tle_ref, tue_ref, out_ref,
          locT_scr, dtT_scr):
    f32 = jnp.float32
    l = pl.program_id(0)

    @pl.when(l == 0)
    def _():
        i_l = (jax.lax.broadcasted_iota(jnp.int32, (L, L), 0) ==
               jax.lax.broadcasted_iota(jnp.int32, (L, L), 1)).astype(f32)
        locT_scr[...] = jax.lax.dot_general(
            i_l, loc_ref[...].astype(f32), (((1,), (1,)), ((), ())),
            preferred_element_type=f32)
        dtT_scr[...] = jax.lax.dot_general(
            i_l, dt_ref[...], (((1,), (1,)), ((), ())),
            preferred_element_type=f32)

    loc = locT_scr[pl.ds(l, 1), :]
    dt = dtT_scr[pl.ds(l, 1), :]
    valid = jnp.where(l < tl_ref[...], f32(1.0), f32(0.0))

    row = jax.lax.broadcasted_iota(jnp.int32, (ML, B), 0).astype(f32)
    ohT = jnp.where(loc - f32(1.0) == row, f32(1.0), f32(0.0)) * valid

    dsT = _tdot(mat2_ref[...], ohT)

    j_m = jax.lax.broadcasted_iota(jnp.int32, (ML * E, ML), 0) // E
    m_col = jax.lax.broadcasted_iota(jnp.int32, (ML * E, ML), 1)
    MT = (j_m == m_col).astype(f32)
    j_e = jax.lax.broadcasted_iota(jnp.int32, (ML * E, E), 0) % E
    e_col = jax.lax.broadcasted_iota(jnp.int32, (ML * E, E), 1)
    PT = (j_e == e_col).astype(f32)

    i_e = (jax.lax.broadcasted_iota(jnp.int32, (E, E), 0) ==
           jax.lax.broadcasted_iota(jnp.int32, (E, E), 1)).astype(f32)

    def sel(ref):
        t = jax.lax.dot_general(i_e, ref[...], (((1,), (1,)), ((), ())),
                                preferred_element_type=f32)
        return t[:, 1:2] * valid + t[:, 0:1] * (f32(1.0) - valid)

    s_sl = sel(sl_ref)
    s_su = sel(su_ref)
    s_tl = sel(tle_ref)
    s_tu = sel(tue_ref)

    cT = (s_su - s_sl) * f32(1.0 / (ex_su - ex_sl))
    aT = s_sl + (s_tl * (ex_tu - dt) + s_tu * (dt - ex_tl)) * f32(
        1.0 / (ex_tu - ex_tl))

    ds_exp = jnp.dot(MT, dsT, preferred_element_type=f32)
    a_exp = jnp.dot(PT, aT, preferred_element_type=f32)
    c_exp = jnp.dot(PT, cT, preferred_element_type=f32)
    out_ref[...] = a_exp + ds_exp * c_exp


def kernel(traj_location, mat2, vector, traj_length, emb_su, emb_sl, emb_tu, emb_tl):
    B, L = traj_location.shape
    ML = mat2.shape[0]
    E = emb_su.shape[1]
    EX_SU, EX_SL, EX_TU, EX_TL = 100.0, 0.0, 24.0, 0.0

    tl_row = traj_length.reshape(1, B).astype(jnp.int32)

    body = functools.partial(_body, B, L, ML, E, EX_SU, EX_SL, EX_TU, EX_TL)
    full = lambda shape: pl.BlockSpec(shape, lambda l: (0, 0))

    out = pl.pallas_call(
        body,
        grid=(L,),
        in_specs=[
            full((B, L)),
            full((B, L)),
            full((1, B)),
            full((ML, ML)),
            full((2, E)),
            full((2, E)),
            full((2, E)),
            full((2, E)),
        ],
        out_specs=pl.BlockSpec((ML * E, B), lambda l: (l, 0)),
        out_shape=jax.ShapeDtypeStruct((L * ML * E, B), jnp.float32),
        scratch_shapes=[
            pltpu.VMEM((L, B), jnp.float32),
            pltpu.VMEM((L, B), jnp.float32),
        ],
    )(traj_location.astype(jnp.int32), vector, tl_row, mat2,
      emb_su, emb_sl, emb_tl, emb_tu)

    return out.reshape(L, ML, E, B).transpose(3, 0, 1, 2)


# speedup vs baseline: 331.5024x; 1.2198x over previous
import functools

import jax
import jax.numpy as jnp
from jax.experimental import pallas as pl
from jax.experimental.pallas import tpu as pltpu


def _tdot(a, b):
    return jax.lax.dot_general(a, b, (((0,), (0,)), ((), ())),
                               preferred_element_type=jnp.float32)


def _body(B, L, ML, E, ex_su, ex_sl, ex_tu, ex_tl,
          loc_ref, dt_ref, tl_ref, mat2_ref,
          su_ref, sl_ref, tle_ref, tue_ref, out_ref,
          locT_scr, dtT_scr):
    f32 = jnp.float32
    l = pl.program_id(0)

    @pl.when(l == 0)
    def _():
        i_l = (jax.lax.broadcasted_iota(jnp.int32, (L, L), 0) ==
               jax.lax.broadcasted_iota(jnp.int32, (L, L), 1)).astype(f32)
        locT_scr[...] = jax.lax.dot_general(
            i_l, loc_ref[...].astype(f32), (((1,), (1,)), ((), ())),
            preferred_element_type=f32)
        dtT_scr[...] = jax.lax.dot_general(
            i_l, dt_ref[...], (((1,), (1,)), ((), ())),
            preferred_element_type=f32)

    loc = locT_scr[pl.ds(l, 1), :]
    dt = dtT_scr[pl.ds(l, 1), :]
    valid = jnp.where(l < tl_ref[...], f32(1.0), f32(0.0))

    row = jax.lax.broadcasted_iota(jnp.int32, (ML, B), 0).astype(f32)
    ohT = jnp.where(loc - f32(1.0) == row, f32(1.0), f32(0.0)) * valid

    dsT = _tdot(mat2_ref[...], ohT)

    i_e = (jax.lax.broadcasted_iota(jnp.int32, (E, E), 0) ==
           jax.lax.broadcasted_iota(jnp.int32, (E, E), 1)).astype(f32)

    def sel(ref):
        t = jax.lax.dot_general(i_e, ref[...], (((1,), (1,)), ((), ())),
                                preferred_element_type=f32)
        return t[:, 1:2] * valid + t[:, 0:1] * (f32(1.0) - valid)

    s_sl = sel(sl_ref)
    s_su = sel(su_ref)
    s_tl = sel(tle_ref)
    s_tu = sel(tue_ref)

    cT = (s_su - s_sl) * f32(1.0 / (ex_su - ex_sl))
    aT = s_sl + (s_tl * (ex_tu - dt) + s_tu * (dt - ex_tl)) * f32(
        1.0 / (ex_tu - ex_tl))

    ds_b = jnp.broadcast_to(dsT[:, None, :], (ML, E, B))
    a_b = jnp.broadcast_to(aT[None, :, :], (ML, E, B))
    c_b = jnp.broadcast_to(cT[None, :, :], (ML, E, B))
    out_ref[...] = (a_b + ds_b * c_b).reshape(ML * E, B)


def kernel(traj_location, mat2, vector, traj_length, emb_su, emb_sl, emb_tu, emb_tl):
    B, L = traj_location.shape
    ML = mat2.shape[0]
    E = emb_su.shape[1]
    EX_SU, EX_SL, EX_TU, EX_TL = 100.0, 0.0, 24.0, 0.0

    tl_row = traj_length.reshape(1, B).astype(jnp.int32)

    body = functools.partial(_body, B, L, ML, E, EX_SU, EX_SL, EX_TU, EX_TL)
    full = lambda shape: pl.BlockSpec(shape, lambda l: (0, 0))

    out = pl.pallas_call(
        body,
        grid=(L,),
        in_specs=[
            full((B, L)),
            full((B, L)),
            full((1, B)),
            full((ML, ML)),
            full((2, E)),
            full((2, E)),
            full((2, E)),
            full((2, E)),
        ],
        out_specs=pl.BlockSpec((ML * E, B), lambda l: (l, 0)),
        out_shape=jax.ShapeDtypeStruct((L * ML * E, B), jnp.float32),
        scratch_shapes=[
            pltpu.VMEM((L, B), jnp.float32),
            pltpu.VMEM((L, B), jnp.float32),
        ],
    )(traj_location.astype(jnp.int32), vector, tl_row, mat2,
      emb_su, emb_sl, emb_tl, emb_tu)

    return out.reshape(L, ML, E, B).transpose(3, 0, 1, 2)


# speedup vs baseline: 437.6469x; 1.3202x over previous
import functools

import jax
import jax.numpy as jnp
from jax.experimental import pallas as pl
from jax.experimental.pallas import tpu as pltpu


def _tdot(a, b):
    return jax.lax.dot_general(a, b, (((0,), (0,)), ((), ())),
                               preferred_element_type=jnp.float32)


def _body(B, L, ML, E, G, ex_su, ex_sl, ex_tu, ex_tl,
          loc_ref, dt_ref, tl_ref, mat2_ref,
          su_ref, sl_ref, tle_ref, tue_ref, out_ref,
          locT_scr, dtT_scr):
    f32 = jnp.float32
    step = pl.program_id(0)

    @pl.when(step == 0)
    def _():
        i_l = (jax.lax.broadcasted_iota(jnp.int32, (L, L), 0) ==
               jax.lax.broadcasted_iota(jnp.int32, (L, L), 1)).astype(f32)
        locT_scr[...] = jax.lax.dot_general(
            i_l, loc_ref[...].astype(f32), (((1,), (1,)), ((), ())),
            preferred_element_type=f32)
        dtT_scr[...] = jax.lax.dot_general(
            i_l, dt_ref[...], (((1,), (1,)), ((), ())),
            preferred_element_type=f32)

    i_e = (jax.lax.broadcasted_iota(jnp.int32, (E, E), 0) ==
           jax.lax.broadcasted_iota(jnp.int32, (E, E), 1)).astype(f32)
    embT = [jax.lax.dot_general(i_e, r[...], (((1,), (1,)), ((), ())),
                                preferred_element_type=f32)
            for r in (sl_ref, su_ref, tle_ref, tue_ref)]

    row = jax.lax.broadcasted_iota(jnp.int32, (ML, B), 0).astype(f32)
    mat2 = mat2_ref[...]
    tl_row = tl_ref[...]

    for g in range(G):
        l = step * G + g
        loc = locT_scr[pl.ds(l, 1), :]
        dt = dtT_scr[pl.ds(l, 1), :]
        valid = jnp.where(l < tl_row, f32(1.0), f32(0.0))

        ohT = jnp.where(loc - f32(1.0) == row, f32(1.0), f32(0.0)) * valid

        dsT = _tdot(mat2, ohT)

        def sel(t):
            return t[:, 1:2] * valid + t[:, 0:1] * (f32(1.0) - valid)

        s_sl, s_su, s_tl, s_tu = (sel(t) for t in embT)

        cT = (s_su - s_sl) * f32(1.0 / (ex_su - ex_sl))
        aT = s_sl + (s_tl * (ex_tu - dt) + s_tu * (dt - ex_tl)) * f32(
            1.0 / (ex_tu - ex_tl))

        ds_b = jnp.broadcast_to(dsT[:, None, :], (ML, E, B))
        a_b = jnp.broadcast_to(aT[None, :, :], (ML, E, B))
        c_b = jnp.broadcast_to(cT[None, :, :], (ML, E, B))
        out_ref[g * ML * E:(g + 1) * ML * E, :] = (
            a_b + ds_b * c_b).reshape(ML * E, B)


def kernel(traj_location, mat2, vector, traj_length, emb_su, emb_sl, emb_tu, emb_tl):
    B, L = traj_location.shape
    ML = mat2.shape[0]
    E = emb_su.shape[1]
    EX_SU, EX_SL, EX_TU, EX_TL = 100.0, 0.0, 24.0, 0.0

    tl_row = traj_length.reshape(1, B).astype(jnp.int32)

    G = 5
    body = functools.partial(_body, B, L, ML, E, G,
                             EX_SU, EX_SL, EX_TU, EX_TL)
    full = lambda shape: pl.BlockSpec(shape, lambda l: (0, 0))

    out = pl.pallas_call(
        body,
        grid=(L // G,),
        in_specs=[
            full((B, L)),
            full((B, L)),
            full((1, B)),
            full((ML, ML)),
            full((2, E)),
            full((2, E)),
            full((2, E)),
            full((2, E)),
        ],
        out_specs=pl.BlockSpec((G * ML * E, B), lambda l: (l, 0)),
        out_shape=jax.ShapeDtypeStruct((L * ML * E, B), jnp.float32),
        scratch_shapes=[
            pltpu.VMEM((L, B), jnp.float32),
            pltpu.VMEM((L, B), jnp.float32),
        ],
    )(traj_location.astype(jnp.int32), vector, tl_row, mat2,
      emb_su, emb_sl, emb_tl, emb_tu)

    return out.reshape(L, ML, E, B).transpose(3, 0, 1, 2)
